# baseline (device time: 131558 ns/iter reference)
import numpy as np

import jax
import jax.numpy as jnp
from jax import lax
from jax.experimental import pallas as pl
from jax.experimental.pallas import tpu as pltpu

N_DEV = 32
N_R = 16
N_L = 15


def _cycle_tables():
    def snake_id(x, y, z):
        return 8 * z + 2 * y + (x if y % 2 == 0 else 1 - x)

    path44 = []
    for z in range(4):
        ys = range(4) if z % 2 == 0 else reversed(range(4))
        path44.extend((y, z) for y in ys)
    cyc = [(0, y, z) for (y, z) in path44]
    cyc += [(1, y, z) for (y, z) in reversed(path44)]
    perm = np.array([snake_id(*c) for c in cyc])
    inv = np.empty(N_DEV, dtype=np.int64)
    inv[perm] = np.arange(N_DEV)

    right_tab = np.empty(N_DEV, dtype=np.int32)
    left_tab = np.empty(N_DEV, dtype=np.int32)
    r_orig_tab = np.empty((N_DEV, N_R), dtype=np.int32)
    l_orig_tab = np.empty((N_DEV, N_L), dtype=np.int32)
    for m in range(N_DEV):
        p = inv[m]
        right_tab[m] = perm[(p + 1) % N_DEV]
        left_tab[m] = perm[(p - 1) % N_DEV]
        for h in range(1, N_R + 1):
            r_orig_tab[m, h - 1] = perm[(p - h) % N_DEV]
        for h in range(1, N_L + 1):
            l_orig_tab[m, h - 1] = perm[(p + h) % N_DEV]
    return right_tab, left_tab, r_orig_tab, l_orig_tab


_RIGHT_TAB, _LEFT_TAB, _R_ORIG_TAB, _L_ORIG_TAB = _cycle_tables()


def kernel(x, w_mat, scale_x, scale_w):
    m_per, k = x.shape
    _, n_per = w_mat.shape

    my = lax.axis_index("i")
    nbrs = jnp.stack(
        [jnp.asarray(_LEFT_TAB)[my], jnp.asarray(_RIGHT_TAB)[my]]
    ).astype(jnp.int32)
    r_orig = jnp.asarray(_R_ORIG_TAB)[my]
    l_orig = jnp.asarray(_L_ORIG_TAB)[my]

    def body(nbrs_ref, r_orig_ref, l_orig_ref,
             x_ref, w_ref, sx_ref, sw_ref, out_ref, xg_ref,
             r_send_a, r_send_b, r_recv_a, r_recv_b, l_send, l_recv):
        def r_sems(h):
            if h % 2 == 0:
                return r_send_a.at[h // 2], r_recv_a.at[h // 2]
            return r_send_b.at[h // 2], r_recv_b.at[h // 2]
        my_id = lax.axis_index("i")
        left = nbrs_ref[0]
        right = nbrs_ref[1]

        barrier_sem = pltpu.get_barrier_semaphore()
        for nbr in (left, right):
            pl.semaphore_signal(
                barrier_sem, inc=1,
                device_id=(nbr,), device_id_type=pl.DeviceIdType.MESH,
            )
        pl.semaphore_wait(barrier_sem, 2)

        def rdma_to(nbr, off, send_sem, recv_sem):
            return pltpu.make_async_remote_copy(
                src_ref=xg_ref.at[pl.ds(off, m_per), :],
                dst_ref=xg_ref.at[pl.ds(off, m_per), :],
                send_sem=send_sem,
                recv_sem=recv_sem,
                device_id=(nbr,),
                device_id_type=pl.DeviceIdType.MESH,
            )

        my_off = pl.multiple_of(my_id * m_per, m_per)
        xg_ref[pl.ds(my_off, m_per), :] = x_ref[:, :].astype(jnp.float8_e5m2)

        rs = [rdma_to(right, my_off, *r_sems(0))]
        ls = [rdma_to(left, my_off, l_send.at[0], l_recv.at[0])]
        rs[0].start()
        ls[0].start()

        w_bf = w_ref[:, :].astype(jnp.bfloat16)
        scale = sx_ref[0] * sw_ref[0]

        def gemm(off):
            rows = pl.ds(off, m_per)
            acc = jnp.dot(
                xg_ref[rows, :].astype(jnp.bfloat16), w_bf,
                preferred_element_type=jnp.float32,
            )
            out_ref[rows, :] = acc * scale

        gemm(my_off)

        sent_waited = set()

        def start_bounded(stream, nxt):
            if len(stream) >= 2:
                stream[-2].wait_send()
                sent_waited.add(id(stream[-2]))
            nxt.start()
            stream.append(nxt)

        for h in range(1, N_R + 1):
            off_r = pl.multiple_of(r_orig_ref[h - 1] * m_per, m_per)
            rs[h - 1].wait_recv()
            if h < N_R:
                start_bounded(rs, rdma_to(right, off_r, *r_sems(h)))

            off_l = None
            if h <= N_L:
                off_l = pl.multiple_of(l_orig_ref[h - 1] * m_per, m_per)
                ls[h - 1].wait()
                if h < N_L:
                    nxt = rdma_to(left, off_l, l_send.at[h], l_recv.at[h])
                    nxt.start()
                    ls.append(nxt)
                    sent_waited.add(id(ls[h - 1]))

            gemm(off_r)
            if off_l is not None:
                gemm(off_l)
        sent_waited.add(id(ls[-1]))

        for d in rs + ls:
            if id(d) not in sent_waited:
                d.wait_send()

    return pl.pallas_call(
        body,
        out_shape=jax.ShapeDtypeStruct((N_DEV * m_per, n_per), jnp.float32),
        in_specs=[
            pl.BlockSpec(memory_space=pltpu.SMEM),
            pl.BlockSpec(memory_space=pltpu.SMEM),
            pl.BlockSpec(memory_space=pltpu.SMEM),
            pl.BlockSpec(memory_space=pltpu.VMEM),
            pl.BlockSpec(memory_space=pltpu.VMEM),
            pl.BlockSpec(memory_space=pltpu.VMEM),
            pl.BlockSpec(memory_space=pltpu.VMEM),
        ],
        out_specs=pl.BlockSpec(memory_space=pltpu.VMEM),
        scratch_shapes=[
            pltpu.VMEM((N_DEV * m_per, k), jnp.float8_e5m2),
            pltpu.SemaphoreType.DMA((N_R // 2,)),
            pltpu.SemaphoreType.DMA((N_R // 2,)),
            pltpu.SemaphoreType.DMA((N_R // 2,)),
            pltpu.SemaphoreType.DMA((N_R // 2,)),
            pltpu.SemaphoreType.DMA((N_L,)),
            pltpu.SemaphoreType.DMA((N_L,)),
        ],
        compiler_params=pltpu.CompilerParams(collective_id=0),
    )(nbrs, r_orig, l_orig, x, w_mat, scale_x, scale_w)


# device time: 131470 ns/iter; 1.0007x vs baseline; 1.0007x over previous
import numpy as np

import jax
import jax.numpy as jnp
from jax import lax
from jax.experimental import pallas as pl
from jax.experimental.pallas import tpu as pltpu

N_DEV = 32
N_R = 16
N_L = 15


def _cycle_tables():
    def snake_id(x, y, z):
        return 8 * z + 2 * y + (x if y % 2 == 0 else 1 - x)

    path44 = []
    for z in range(4):
        ys = range(4) if z % 2 == 0 else reversed(range(4))
        path44.extend((y, z) for y in ys)
    cyc = [(0, y, z) for (y, z) in path44]
    cyc += [(1, y, z) for (y, z) in reversed(path44)]
    perm = np.array([snake_id(*c) for c in cyc])
    inv = np.empty(N_DEV, dtype=np.int64)
    inv[perm] = np.arange(N_DEV)

    right_tab = np.empty(N_DEV, dtype=np.int32)
    left_tab = np.empty(N_DEV, dtype=np.int32)
    r_orig_tab = np.empty((N_DEV, N_R), dtype=np.int32)
    l_orig_tab = np.empty((N_DEV, N_L), dtype=np.int32)
    for m in range(N_DEV):
        p = inv[m]
        right_tab[m] = perm[(p + 1) % N_DEV]
        left_tab[m] = perm[(p - 1) % N_DEV]
        for h in range(1, N_R + 1):
            r_orig_tab[m, h - 1] = perm[(p - h) % N_DEV]
        for h in range(1, N_L + 1):
            l_orig_tab[m, h - 1] = perm[(p + h) % N_DEV]
    return right_tab, left_tab, r_orig_tab, l_orig_tab


_RIGHT_TAB, _LEFT_TAB, _R_ORIG_TAB, _L_ORIG_TAB = _cycle_tables()


def kernel(x, w_mat, scale_x, scale_w):
    m_per, k = x.shape
    _, n_per = w_mat.shape

    my = lax.axis_index("i")
    nbrs = jnp.stack(
        [jnp.asarray(_LEFT_TAB)[my], jnp.asarray(_RIGHT_TAB)[my]]
    ).astype(jnp.int32)
    r_orig = jnp.asarray(_R_ORIG_TAB)[my]
    l_orig = jnp.asarray(_L_ORIG_TAB)[my]

    def body(nbrs_ref, r_orig_ref, l_orig_ref,
             x_ref, w_ref, sx_ref, sw_ref, out_ref, xg_ref,
             r_send_a, r_send_b, r_recv_a, r_recv_b,
             l_send_a, l_send_b, l_recv_a, l_recv_b):
        def r_sems(h):
            if h % 2 == 0:
                return r_send_a.at[h // 2], r_recv_a.at[h // 2]
            return r_send_b.at[h // 2], r_recv_b.at[h // 2]

        def l_sems(h):
            if h % 2 == 0:
                return l_send_a.at[h // 2], l_recv_a.at[h // 2]
            return l_send_b.at[h // 2], l_recv_b.at[h // 2]
        my_id = lax.axis_index("i")
        left = nbrs_ref[0]
        right = nbrs_ref[1]

        barrier_sem = pltpu.get_barrier_semaphore()
        for nbr in (left, right):
            pl.semaphore_signal(
                barrier_sem, inc=1,
                device_id=(nbr,), device_id_type=pl.DeviceIdType.MESH,
            )
        pl.semaphore_wait(barrier_sem, 2)

        def rdma_to(nbr, off, send_sem, recv_sem):
            return pltpu.make_async_remote_copy(
                src_ref=xg_ref.at[pl.ds(off, m_per), :],
                dst_ref=xg_ref.at[pl.ds(off, m_per), :],
                send_sem=send_sem,
                recv_sem=recv_sem,
                device_id=(nbr,),
                device_id_type=pl.DeviceIdType.MESH,
            )

        my_off = pl.multiple_of(my_id * m_per, m_per)
        xg_ref[pl.ds(my_off, m_per), :] = x_ref[:, :].astype(jnp.float8_e5m2)

        rs = [rdma_to(right, my_off, *r_sems(0))]
        ls = [rdma_to(left, my_off, *l_sems(0))]
        rs[0].start()
        ls[0].start()

        w_bf = w_ref[:, :].astype(jnp.bfloat16)
        scale = sx_ref[0] * sw_ref[0]

        def gemm(off):
            rows = pl.ds(off, m_per)
            acc = jnp.dot(
                xg_ref[rows, :].astype(jnp.bfloat16), w_bf,
                preferred_element_type=jnp.float32,
            )
            out_ref[rows, :] = acc * scale

        gemm(my_off)

        sent_waited = set()

        def start_bounded(stream, nxt):
            if len(stream) >= 2:
                stream[-2].wait_send()
                sent_waited.add(id(stream[-2]))
            nxt.start()
            stream.append(nxt)

        for h in range(1, N_R + 1):
            off_r = pl.multiple_of(r_orig_ref[h - 1] * m_per, m_per)
            rs[h - 1].wait_recv()
            if h < N_R:
                start_bounded(rs, rdma_to(right, off_r, *r_sems(h)))

            off_l = None
            if h <= N_L:
                off_l = pl.multiple_of(l_orig_ref[h - 1] * m_per, m_per)
                ls[h - 1].wait_recv()
                if h < N_L:
                    start_bounded(ls, rdma_to(left, off_l, *l_sems(h)))

            gemm(off_r)
            if off_l is not None:
                gemm(off_l)

        for d in rs + ls:
            if id(d) not in sent_waited:
                d.wait_send()

    return pl.pallas_call(
        body,
        out_shape=jax.ShapeDtypeStruct((N_DEV * m_per, n_per), jnp.float32),
        in_specs=[
            pl.BlockSpec(memory_space=pltpu.SMEM),
            pl.BlockSpec(memory_space=pltpu.SMEM),
            pl.BlockSpec(memory_space=pltpu.SMEM),
            pl.BlockSpec(memory_space=pltpu.VMEM),
            pl.BlockSpec(memory_space=pltpu.VMEM),
            pl.BlockSpec(memory_space=pltpu.VMEM),
            pl.BlockSpec(memory_space=pltpu.VMEM),
        ],
        out_specs=pl.BlockSpec(memory_space=pltpu.VMEM),
        scratch_shapes=[
            pltpu.VMEM((N_DEV * m_per, k), jnp.float8_e5m2),
            pltpu.SemaphoreType.DMA((N_R // 2,)),
            pltpu.SemaphoreType.DMA((N_R // 2,)),
            pltpu.SemaphoreType.DMA((N_R // 2,)),
            pltpu.SemaphoreType.DMA((N_R // 2,)),
            pltpu.SemaphoreType.DMA((N_L // 2 + 1,)),
            pltpu.SemaphoreType.DMA((N_L // 2,)),
            pltpu.SemaphoreType.DMA((N_L // 2 + 1,)),
            pltpu.SemaphoreType.DMA((N_L // 2,)),
        ],
        compiler_params=pltpu.CompilerParams(collective_id=0),
    )(nbrs, r_orig, l_orig, x, w_mat, scale_x, scale_w)


# device time: 105280 ns/iter; 1.2496x vs baseline; 1.2488x over previous
import numpy as np

import jax
import jax.numpy as jnp
from jax import lax
from jax.experimental import pallas as pl
from jax.experimental.pallas import tpu as pltpu

N_DEV = 32
N_R = 16
N_L = 15
SUB = 2
R_SENDS, R_RECVS = SUB * N_R, SUB * N_R
L_SENDS, L_RECVS = SUB * N_L, SUB * N_L


def _cycle_tables():
    def snake_id(x, y, z):
        return 8 * z + 2 * y + (x if y % 2 == 0 else 1 - x)

    path44 = []
    for z in range(4):
        ys = range(4) if z % 2 == 0 else reversed(range(4))
        path44.extend((y, z) for y in ys)
    cyc = [(0, y, z) for (y, z) in path44]
    cyc += [(1, y, z) for (y, z) in reversed(path44)]
    perm = np.array([snake_id(*c) for c in cyc])
    inv = np.empty(N_DEV, dtype=np.int64)
    inv[perm] = np.arange(N_DEV)

    right_tab = np.empty(N_DEV, dtype=np.int32)
    left_tab = np.empty(N_DEV, dtype=np.int32)
    r_orig_tab = np.empty((N_DEV, N_R), dtype=np.int32)
    l_orig_tab = np.empty((N_DEV, N_L), dtype=np.int32)
    for m in range(N_DEV):
        p = inv[m]
        right_tab[m] = perm[(p + 1) % N_DEV]
        left_tab[m] = perm[(p - 1) % N_DEV]
        for h in range(1, N_R + 1):
            r_orig_tab[m, h - 1] = perm[(p - h) % N_DEV]
        for h in range(1, N_L + 1):
            l_orig_tab[m, h - 1] = perm[(p + h) % N_DEV]
    return right_tab, left_tab, r_orig_tab, l_orig_tab


_RIGHT_TAB, _LEFT_TAB, _R_ORIG_TAB, _L_ORIG_TAB = _cycle_tables()


def kernel(x, w_mat, scale_x, scale_w):
    m_per, k = x.shape
    _, n_per = w_mat.shape
    m_sub = m_per // SUB

    my = lax.axis_index("i")
    nbrs = jnp.stack(
        [jnp.asarray(_LEFT_TAB)[my], jnp.asarray(_RIGHT_TAB)[my]]
    ).astype(jnp.int32)
    r_orig = jnp.asarray(_R_ORIG_TAB)[my]
    l_orig = jnp.asarray(_L_ORIG_TAB)[my]

    def body(nbrs_ref, r_orig_ref, l_orig_ref,
             x_ref, w_ref, sx_ref, sw_ref, out_ref, xg_ref,
             r_send_a, r_send_b, r_recv_a, r_recv_b,
             l_send_a, l_send_b, l_recv_a, l_recv_b):
        def r_sems(j):
            if j % 2 == 0:
                return r_send_a.at[j // 2], r_recv_a.at[j // 2]
            return r_send_b.at[j // 2], r_recv_b.at[j // 2]

        def l_sems(j):
            if j % 2 == 0:
                return l_send_a.at[j // 2], l_recv_a.at[j // 2]
            return l_send_b.at[j // 2], l_recv_b.at[j // 2]

        my_id = lax.axis_index("i")
        left = nbrs_ref[0]
        right = nbrs_ref[1]

        barrier_sem = pltpu.get_barrier_semaphore()
        for nbr in (left, right):
            pl.semaphore_signal(
                barrier_sem, inc=1,
                device_id=(nbr,), device_id_type=pl.DeviceIdType.MESH,
            )
        pl.semaphore_wait(barrier_sem, 2)

        def rdma_to(nbr, off, send_sem, recv_sem):
            return pltpu.make_async_remote_copy(
                src_ref=xg_ref.at[pl.ds(off, m_sub), :],
                dst_ref=xg_ref.at[pl.ds(off, m_sub), :],
                send_sem=send_sem,
                recv_sem=recv_sem,
                device_id=(nbr,),
                device_id_type=pl.DeviceIdType.MESH,
            )

        my_off = pl.multiple_of(my_id * m_per, m_per)
        xg_ref[pl.ds(my_off, m_per), :] = x_ref[:, :].astype(jnp.float8_e5m2)

        w_bf = w_ref[:, :].astype(jnp.bfloat16)
        scale = sx_ref[0] * sw_ref[0]

        def gemm(off):
            rows = pl.ds(off, m_per)
            acc = jnp.dot(
                xg_ref[rows, :].astype(jnp.bfloat16), w_bf,
                preferred_element_type=jnp.float32,
            )
            out_ref[rows, :] = acc * scale

        sent_waited = set()

        def start_bounded(stream, nxt):
            if len(stream) >= 2:
                stream[-2].wait_send()
                sent_waited.add(id(stream[-2]))
            nxt.start()
            stream.append(nxt)

        rs, ls = [], []
        for s in range(SUB):
            off = pl.multiple_of(my_id * m_per + s * m_sub, m_sub)
            start_bounded(rs, rdma_to(right, off, *r_sems(s)))
            start_bounded(ls, rdma_to(left, off, *l_sems(s)))

        gemm(my_off)

        for j in range(2, R_RECVS + 2):
            i = j - 2
            d, s = i // 2 + 1, i % 2
            off_r = pl.multiple_of(
                r_orig_ref[d - 1] * m_per + s * m_sub, m_sub
            )
            rs[i].wait_recv()
            if j < R_SENDS:
                start_bounded(rs, rdma_to(right, off_r, *r_sems(j)))
            if s == 1:
                gemm(pl.multiple_of(r_orig_ref[d - 1] * m_per, m_per))

            if i < L_RECVS:
                off_l = pl.multiple_of(
                    l_orig_ref[d - 1] * m_per + s * m_sub, m_sub
                )
                ls[i].wait_recv()
                if j < L_SENDS:
                    start_bounded(ls, rdma_to(left, off_l, *l_sems(j)))
                if s == 1:
                    gemm(pl.multiple_of(l_orig_ref[d - 1] * m_per, m_per))

        for dsc in rs + ls:
            if id(dsc) not in sent_waited:
                dsc.wait_send()

    return pl.pallas_call(
        body,
        out_shape=jax.ShapeDtypeStruct((N_DEV * m_per, n_per), jnp.float32),
        in_specs=[
            pl.BlockSpec(memory_space=pltpu.SMEM),
            pl.BlockSpec(memory_space=pltpu.SMEM),
            pl.BlockSpec(memory_space=pltpu.SMEM),
            pl.BlockSpec(memory_space=pltpu.VMEM),
            pl.BlockSpec(memory_space=pltpu.VMEM),
            pl.BlockSpec(memory_space=pltpu.VMEM),
            pl.BlockSpec(memory_space=pltpu.VMEM),
        ],
        out_specs=pl.BlockSpec(memory_space=pltpu.VMEM),
        scratch_shapes=[
            pltpu.VMEM((N_DEV * m_per, k), jnp.float8_e5m2),
            pltpu.SemaphoreType.DMA((R_SENDS // 2,)),
            pltpu.SemaphoreType.DMA((R_SENDS // 2,)),
            pltpu.SemaphoreType.DMA((R_RECVS // 2,)),
            pltpu.SemaphoreType.DMA((R_RECVS // 2,)),
            pltpu.SemaphoreType.DMA((L_SENDS // 2,)),
            pltpu.SemaphoreType.DMA((L_SENDS // 2,)),
            pltpu.SemaphoreType.DMA((L_RECVS // 2,)),
            pltpu.SemaphoreType.DMA((L_RECVS // 2,)),
        ],
        compiler_params=pltpu.CompilerParams(collective_id=0),
    )(nbrs, r_orig, l_orig, x, w_mat, scale_x, scale_w)
